# baseline (speedup 1.0000x reference)
import functools

import jax
import jax.numpy as jnp
from jax.experimental import pallas as pl
from jax.experimental.pallas import tpu as pltpu

K = 64
NITER = 25
N = 16384
DIM = 128
CHUNK = 4096
TILE = 512
NCH = N // CHUNK
TPC = CHUNK // TILE
NTILE = N // TILE
DEFAULT = jax.lax.Precision.DEFAULT
HIGHEST = jax.lax.Precision.HIGHEST

_EH = [720, 1440, 2160] + list(range(2640, 7921, 480)) + [8192]
WINDOW_ENDS = _EH + [8192 + e for e in _EH[:-1]]


def _dot(a, b, dims, precision):
    return jax.lax.dot_general(
        a, b, dimension_numbers=(dims, ((), ())),
        precision=precision, preferred_element_type=jnp.float32)


def _rowsum_sq(x):
    sq = x * x
    acc8 = sq[:, 0:8]
    for g in range(1, 16):
        acc8 = acc8 + sq[:, 8 * g:8 * g + 8]
    a4 = acc8[:, 0:4] + acc8[:, 4:8]
    a2 = a4[:, 0:2] + a4[:, 2:4]
    return a2[:, 0:1] + a2[:, 1:2]


def _c2_row(cent, eye_k):
    c2_col = _rowsum_sq(cent)
    return _dot(c2_col, eye_k, ((0,), (0,)), HIGHEST)


def _argmin_col(d, iota_k):
    dmin = jnp.min(d, axis=1, keepdims=True)
    return jnp.min(jnp.where(d == dmin, iota_k, K), axis=1, keepdims=True)


def _argmax_col(d, iota_k):
    dmax = jnp.max(d, axis=1, keepdims=True)
    return jnp.min(jnp.where(d == dmax, iota_k, K), axis=1, keepdims=True)


def _centroids(sums, counts_col, cent_prev):
    return jnp.where(counts_col > 0.0,
                     sums / jnp.maximum(counts_col, 1.0), cent_prev)


def _assign_kernel(emb_ref, sums_ref, counts_ref, cent_prev_ref, wrow_ref,
                   packed_ref, counts_out_ref, cent_out_ref,
                   amin_ref, tilecnt_ref):
    cent = _centroids(sums_ref[...], counts_ref[...], cent_prev_ref[...])
    cent_out_ref[...] = cent
    iota_k = jax.lax.broadcasted_iota(jnp.int32, (1, K), 1)
    iota_kc = jax.lax.broadcasted_iota(jnp.int32, (TILE, K), 1)
    ey_r = jax.lax.broadcasted_iota(jnp.int32, (K, K), 0)
    ey_c = jax.lax.broadcasted_iota(jnp.int32, (K, K), 1)
    eye_k = (ey_r == ey_c).astype(jnp.float32)
    c2 = _c2_row(cent, eye_k)

    r_i = jax.lax.broadcasted_iota(jnp.int32, (TILE, TILE), 0)
    c_i = jax.lax.broadcasted_iota(jnp.int32, (TILE, TILE), 1)
    tri = (c_i < r_i).astype(jnp.float32)

    sel_t = jax.lax.broadcasted_iota(jnp.int32, (TPC, CHUNK), 0)
    sel_r = jax.lax.broadcasted_iota(jnp.int32, (TPC, CHUNK), 1)
    tile_sel = (sel_r // TILE == sel_t).astype(jnp.float32)

    def pass1(j, counts_row):
        x = emb_ref[pl.ds(j * CHUNK, CHUNK), :]
        x2c = _rowsum_sq(x)
        d = x2c + c2 - 2.0 * _dot(x, cent, ((1,), (1,)), DEFAULT)
        amin = _argmin_col(d, iota_k)
        amin_ref[pl.ds(j * CHUNK, CHUNK), :] = amin
        onehot = (amin == iota_k).astype(jnp.float32)
        counts_row = counts_row + _dot(
            jnp.ones((1, CHUNK), jnp.float32), onehot, ((1,), (0,)), DEFAULT)
        tilecnt_ref[pl.ds(j * TPC, TPC), :] = _dot(
            tile_sel, onehot, ((1,), (0,)), DEFAULT)
        return counts_row

    counts_row = jax.lax.fori_loop(0, NCH, pass1,
                                   jnp.zeros((1, K), jnp.float32))
    counts_out_ref[...] = counts_row

    cs_r = jax.lax.broadcasted_iota(jnp.int32, (K, K), 0)
    cs_c = jax.lax.broadcasted_iota(jnp.int32, (K, K), 1)
    triu = (cs_r < cs_c).astype(jnp.float32)
    offsets = _dot(counts_row, triu, ((1,), (0,)), HIGHEST)

    tr_r = jax.lax.broadcasted_iota(jnp.int32, (NTILE, NTILE), 0)
    tr_c = jax.lax.broadcasted_iota(jnp.int32, (NTILE, NTILE), 1)
    tri_t = (tr_c < tr_r).astype(jnp.float32)
    tile_prefix = _dot(tri_t, tilecnt_ref[...], ((1,), (0,)), HIGHEST)
    tilecnt_ref[...] = tile_prefix

    w_row = wrow_ref[...]

    def pass2(t, _):
        amin = amin_ref[pl.ds(t * TILE, TILE), :]
        onehot = (amin == iota_kc).astype(jnp.float32)
        lrank = _dot(tri, onehot, ((1,), (0,)), DEFAULT)
        base = tilecnt_ref[pl.ds(t, 1), :]
        posf = lrank + base + offsets
        pos = jnp.sum(posf * onehot, axis=1, keepdims=True)
        flag = (jnp.sum((pos == w_row).astype(jnp.float32), axis=1,
                        keepdims=True) > 0.0).astype(jnp.int32)
        packed_ref[pl.ds(t * TILE, TILE), :] = amin + 128 * flag
        return 0

    jax.lax.fori_loop(0, NTILE, pass2, 0)


def _scatter_kernel(packed_ref, emb_ref, sums_ref, partial_ref, total_ref):
    g = pl.program_id(0)

    @pl.when(g == 0)
    def _init():
        partial_ref[...] = jnp.zeros((K, DIM), jnp.float32)
        total_ref[...] = jnp.zeros((K, DIM), jnp.float32)

    def body(i, _):
        v = packed_ref[0, 0, i]
        s = jax.lax.rem(v, 128)
        f = jax.lax.div(v, 128)

        @pl.when(f == 1)
        def _flush():
            total_ref[pl.ds(s, 1), :] = (total_ref[pl.ds(s, 1), :]
                                         + partial_ref[pl.ds(s, 1), :])
            partial_ref[pl.ds(s, 1), :] = jnp.zeros((1, DIM), jnp.float32)

        partial_ref[pl.ds(s, 1), :] = (partial_ref[pl.ds(s, 1), :]
                                       + emb_ref[pl.ds(i, 1), :])
        return 0

    jax.lax.fori_loop(0, CHUNK, body, 0)

    @pl.when(g == NCH - 1)
    def _fin():
        sums_ref[...] = total_ref[...] + partial_ref[...]


def _final_kernel(emb_ref, sums_ref, counts_ref, cent_prev_ref, out_ref):
    cent = _centroids(sums_ref[...], counts_ref[...], cent_prev_ref[...])
    iota_k = jax.lax.broadcasted_iota(jnp.int32, (1, K), 1)
    ey_r = jax.lax.broadcasted_iota(jnp.int32, (K, K), 0)
    ey_c = jax.lax.broadcasted_iota(jnp.int32, (K, K), 1)
    eye_k = (ey_r == ey_c).astype(jnp.float32)
    c2 = _c2_row(cent, eye_k)

    def chunk(j, _):
        x = emb_ref[pl.ds(j * CHUNK, CHUNK), :]
        x2c = _rowsum_sq(x)
        d = x2c + c2 - 2.0 * _dot(x, cent, ((1,), (1,)), DEFAULT)
        s = jnp.sqrt(jnp.maximum(d, 0.0))
        i1 = _argmax_col(s, iota_k)
        s_masked = jnp.where(iota_k == i1, -jnp.inf, s)
        i2 = _argmax_col(s_masked, iota_k)
        sel = (i2 == iota_k).astype(jnp.float32)
        out_ref[pl.ds(j * CHUNK, CHUNK), :] = _dot(
            sel, cent, ((1,), (0,)), HIGHEST)
        return 0

    jax.lax.fori_loop(0, NCH, chunk, 0)


_assign_call_dbg = pl.pallas_call(
        _assign_kernel,
        out_shape=(
            jax.ShapeDtypeStruct((N, 1), jnp.int32),
            jax.ShapeDtypeStruct((1, K), jnp.float32),
            jax.ShapeDtypeStruct((K, DIM), jnp.float32),
        ),
        in_specs=[
            pl.BlockSpec((N, DIM), lambda: (0, 0)),
            pl.BlockSpec((K, DIM), lambda: (0, 0)),
            pl.BlockSpec((K, 1), lambda: (0, 0)),
            pl.BlockSpec((K, DIM), lambda: (0, 0)),
            pl.BlockSpec((1, K), lambda: (0, 0)),
        ],
        out_specs=(
            pl.BlockSpec((N, 1), lambda: (0, 0)),
            pl.BlockSpec((1, K), lambda: (0, 0)),
            pl.BlockSpec((K, DIM), lambda: (0, 0)),
        ),
        scratch_shapes=[
            pltpu.VMEM((N, 1), jnp.int32),
            pltpu.VMEM((NTILE, K), jnp.float32),
        ],
    )

_scatter_call_dbg = pl.pallas_call(
        _scatter_kernel,
        grid=(NCH,),
        out_shape=jax.ShapeDtypeStruct((K, DIM), jnp.float32),
        in_specs=[
            pl.BlockSpec((1, 1, CHUNK), lambda g: (g, 0, 0),
                         memory_space=pltpu.MemorySpace.SMEM),
            pl.BlockSpec((CHUNK, DIM), lambda g: (g, 0)),
        ],
        out_specs=pl.BlockSpec((K, DIM), lambda g: (0, 0)),
        scratch_shapes=[
            pltpu.VMEM((K, DIM), jnp.float32),
            pltpu.VMEM((K, DIM), jnp.float32),
        ],
    )

_final_call_dbg = pl.pallas_call(
        _final_kernel,
        out_shape=jax.ShapeDtypeStruct((N, DIM), jnp.float32),
        in_specs=[
            pl.BlockSpec((N, DIM), lambda: (0, 0)),
            pl.BlockSpec((K, DIM), lambda: (0, 0)),
            pl.BlockSpec((K, 1), lambda: (0, 0)),
            pl.BlockSpec((K, DIM), lambda: (0, 0)),
        ],
        out_specs=pl.BlockSpec((N, DIM), lambda: (0, 0)),
    )

_W_ROW = jnp.array(WINDOW_ENDS + [-1] * (K - len(WINDOW_ENDS)),
                   jnp.float32).reshape(1, K)


@functools.partial(jax.jit, static_argnames=())
def kernel(embeddings, batch_id):
    del batch_id
    emb = embeddings
    w_row = _W_ROW
    assign_call = _assign_call_dbg
    scatter_call = _scatter_call_dbg
    final_call = _final_call_dbg

    def body(_, carry):
        sums, counts_col, cent = carry
        packed, counts_row, cent_new = assign_call(
            emb, sums, counts_col, cent, w_row)
        sums_new = scatter_call(packed.reshape(NCH, 1, CHUNK), emb)
        return sums_new, counts_row.reshape(K, 1), cent_new

    sums0 = emb[:K]
    counts0 = jnp.ones((K, 1), jnp.float32)
    cent0 = jnp.zeros((K, DIM), jnp.float32)
    sums, counts_col, cent = jax.lax.fori_loop(
        0, NITER, body, (sums0, counts0, cent0))

    return final_call(emb, sums, counts_col, cent)


# speedup vs baseline: 1.9648x; 1.9648x over previous
import functools

import jax
import jax.numpy as jnp
from jax.experimental import pallas as pl
from jax.experimental.pallas import tpu as pltpu

K = 64
NITER = 25
N = 16384
DIM = 128
CHUNK = 4096
TILE = 512
NCH = N // CHUNK
TPC = CHUNK // TILE
NTILE = N // TILE
DEFAULT = jax.lax.Precision.DEFAULT
HIGHEST = jax.lax.Precision.HIGHEST

_EH = [720, 1440, 2160] + list(range(2640, 7921, 480)) + [8192]
WINDOW_ENDS = _EH + [8192 + e for e in _EH[:-1]]


def _dot(a, b, dims, precision):
    return jax.lax.dot_general(
        a, b, dimension_numbers=(dims, ((), ())),
        precision=precision, preferred_element_type=jnp.float32)


def _rowsum_sq(x):
    sq = x * x
    acc8 = sq[:, 0:8]
    for g in range(1, 16):
        acc8 = acc8 + sq[:, 8 * g:8 * g + 8]
    a4 = acc8[:, 0:4] + acc8[:, 4:8]
    a2 = a4[:, 0:2] + a4[:, 2:4]
    return a2[:, 0:1] + a2[:, 1:2]


def _c2_row(cent, eye_k):
    c2_col = _rowsum_sq(cent)
    return _dot(c2_col, eye_k, ((0,), (0,)), HIGHEST)


def _argmin_col(d, iota_k):
    dmin = jnp.min(d, axis=1, keepdims=True)
    return jnp.min(jnp.where(d == dmin, iota_k, K), axis=1, keepdims=True)


def _argmax_col(d, iota_k):
    dmax = jnp.max(d, axis=1, keepdims=True)
    return jnp.min(jnp.where(d == dmax, iota_k, K), axis=1, keepdims=True)


def _centroids(sums, counts_col, cent_prev):
    return jnp.where(counts_col > 0.0,
                     sums / jnp.maximum(counts_col, 1.0), cent_prev)


def _assign_kernel(emb_ref, sums_ref, counts_ref, cent_prev_ref, wrow_ref,
                   packed_ref, counts_out_ref, cent_out_ref, ev_out_ref,
                   amin_ref, tilecnt_ref):
    cent = _centroids(sums_ref[...], counts_ref[...], cent_prev_ref[...])
    cent_out_ref[...] = cent
    iota_k = jax.lax.broadcasted_iota(jnp.int32, (1, K), 1)
    iota_kc = jax.lax.broadcasted_iota(jnp.int32, (TILE, K), 1)
    ey_r = jax.lax.broadcasted_iota(jnp.int32, (K, K), 0)
    ey_c = jax.lax.broadcasted_iota(jnp.int32, (K, K), 1)
    eye_k = (ey_r == ey_c).astype(jnp.float32)
    c2 = _c2_row(cent, eye_k)

    r_i = jax.lax.broadcasted_iota(jnp.int32, (TILE, TILE), 0)
    c_i = jax.lax.broadcasted_iota(jnp.int32, (TILE, TILE), 1)
    tri = (c_i < r_i).astype(jnp.float32)

    sel_t = jax.lax.broadcasted_iota(jnp.int32, (TPC, CHUNK), 0)
    sel_r = jax.lax.broadcasted_iota(jnp.int32, (TPC, CHUNK), 1)
    tile_sel = (sel_r // TILE == sel_t).astype(jnp.float32)

    def pass1(j, counts_row):
        x = emb_ref[pl.ds(j * CHUNK, CHUNK), :]
        x2c = _rowsum_sq(x)
        d = x2c + c2 - 2.0 * _dot(x, cent, ((1,), (1,)), DEFAULT)
        amin = _argmin_col(d, iota_k)
        amin_ref[pl.ds(j * CHUNK, CHUNK), :] = amin
        onehot = (amin == iota_k).astype(jnp.float32)
        counts_row = counts_row + _dot(
            jnp.ones((1, CHUNK), jnp.float32), onehot, ((1,), (0,)), DEFAULT)
        tilecnt_ref[pl.ds(j * TPC, TPC), :] = _dot(
            tile_sel, onehot, ((1,), (0,)), DEFAULT)
        return counts_row

    counts_row = jax.lax.fori_loop(0, NCH, pass1,
                                   jnp.zeros((1, K), jnp.float32))
    counts_out_ref[...] = counts_row

    cs_r = jax.lax.broadcasted_iota(jnp.int32, (K, K), 0)
    cs_c = jax.lax.broadcasted_iota(jnp.int32, (K, K), 1)
    triu = (cs_r < cs_c).astype(jnp.float32)
    offsets = _dot(counts_row, triu, ((1,), (0,)), HIGHEST)

    tr_r = jax.lax.broadcasted_iota(jnp.int32, (NTILE, NTILE), 0)
    tr_c = jax.lax.broadcasted_iota(jnp.int32, (NTILE, NTILE), 1)
    tri_t = (tr_c < tr_r).astype(jnp.float32)
    tile_prefix = _dot(tri_t, tilecnt_ref[...], ((1,), (0,)), HIGHEST)
    tilecnt_ref[...] = tile_prefix

    w_row = wrow_ref[...]
    rowvec = jax.lax.broadcasted_iota(jnp.int32, (1, TILE), 1).astype(jnp.float32)

    def pass2(t, ev_rows):
        amin = amin_ref[pl.ds(t * TILE, TILE), :]
        onehot = (amin == iota_kc).astype(jnp.float32)
        lrank = _dot(tri, onehot, ((1,), (0,)), DEFAULT)
        base = tilecnt_ref[pl.ds(t, 1), :]
        posf = lrank + base + offsets
        pos = jnp.sum(posf * onehot, axis=1, keepdims=True)
        packed_ref[pl.ds(t * TILE, TILE), :] = amin
        match = (pos == w_row).astype(jnp.float32)
        tf = jnp.float32(t * TILE)
        ev_rows = ev_rows + _dot(rowvec + tf, match, ((1,), (0,)), HIGHEST)
        return ev_rows

    ev_rows = jax.lax.fori_loop(0, NTILE, pass2,
                                jnp.zeros((1, K), jnp.float32))

    k_lane = jax.lax.broadcasted_iota(jnp.int32, (1, K), 1).astype(jnp.float32)
    is_pad = (w_row < 0.0).astype(jnp.float32)
    ev_rows = ev_rows * (1.0 - is_pad) + (100000.0 + k_lane) * is_pad
    evk = jnp.broadcast_to(ev_rows, (K, K))
    ev_diag_col = jnp.sum(evk * eye_k, axis=1, keepdims=True)
    cmp = (evk < ev_diag_col).astype(jnp.float32)
    rank_col = _dot(cmp, jnp.ones((K, 1), jnp.float32), ((1,), (0,)), DEFAULT)
    place = (rank_col == k_lane).astype(jnp.float32)
    ev_sorted = _dot(ev_rows, place, ((1,), (0,)), HIGHEST)
    ev_out_ref[...] = ev_sorted.astype(jnp.int32)


def _scatter_kernel(packed_ref, ev_ref, emb_ref, sums_ref,
                    partial_ref, total_ref):
    g = pl.program_id(0)

    @pl.when(g == 0)
    def _init():
        partial_ref[...] = jnp.zeros((K, DIM), jnp.float32)
        total_ref[...] = jnp.zeros((K, DIM), jnp.float32)

    def row_body(i, _):
        s = packed_ref[0, 0, i]
        partial_ref[pl.ds(s, 1), :] = (partial_ref[pl.ds(s, 1), :]
                                       + emb_ref[pl.ds(i, 1), :])
        return 0

    def ev_body(e, ptr):
        er = ev_ref[0, e] - g * CHUNK
        hi = jnp.clip(er, ptr, CHUNK)
        jax.lax.fori_loop(ptr, hi, row_body, 0)

        @pl.when((er >= ptr) & (er < CHUNK))
        def _flush():
            s = packed_ref[0, 0, er]
            total_ref[pl.ds(s, 1), :] = (total_ref[pl.ds(s, 1), :]
                                         + partial_ref[pl.ds(s, 1), :])
            partial_ref[pl.ds(s, 1), :] = jnp.zeros((1, DIM), jnp.float32)

        return hi

    ptr = jax.lax.fori_loop(0, len(WINDOW_ENDS), ev_body, 0)
    jax.lax.fori_loop(ptr, CHUNK, row_body, 0)

    @pl.when(g == NCH - 1)
    def _fin():
        sums_ref[...] = total_ref[...] + partial_ref[...]


def _final_kernel(emb_ref, sums_ref, counts_ref, cent_prev_ref, out_ref):
    cent = _centroids(sums_ref[...], counts_ref[...], cent_prev_ref[...])
    iota_k = jax.lax.broadcasted_iota(jnp.int32, (1, K), 1)
    ey_r = jax.lax.broadcasted_iota(jnp.int32, (K, K), 0)
    ey_c = jax.lax.broadcasted_iota(jnp.int32, (K, K), 1)
    eye_k = (ey_r == ey_c).astype(jnp.float32)
    c2 = _c2_row(cent, eye_k)

    def chunk(j, _):
        x = emb_ref[pl.ds(j * CHUNK, CHUNK), :]
        x2c = _rowsum_sq(x)
        d = x2c + c2 - 2.0 * _dot(x, cent, ((1,), (1,)), DEFAULT)
        s = jnp.sqrt(jnp.maximum(d, 0.0))
        i1 = _argmax_col(s, iota_k)
        s_masked = jnp.where(iota_k == i1, -jnp.inf, s)
        i2 = _argmax_col(s_masked, iota_k)
        sel = (i2 == iota_k).astype(jnp.float32)
        out_ref[pl.ds(j * CHUNK, CHUNK), :] = _dot(
            sel, cent, ((1,), (0,)), HIGHEST)
        return 0

    jax.lax.fori_loop(0, NCH, chunk, 0)


_assign_call_dbg = pl.pallas_call(
        _assign_kernel,
        out_shape=(
            jax.ShapeDtypeStruct((N, 1), jnp.int32),
            jax.ShapeDtypeStruct((1, K), jnp.float32),
            jax.ShapeDtypeStruct((K, DIM), jnp.float32),
            jax.ShapeDtypeStruct((1, K), jnp.int32),
        ),
        in_specs=[
            pl.BlockSpec((N, DIM), lambda: (0, 0)),
            pl.BlockSpec((K, DIM), lambda: (0, 0)),
            pl.BlockSpec((K, 1), lambda: (0, 0)),
            pl.BlockSpec((K, DIM), lambda: (0, 0)),
            pl.BlockSpec((1, K), lambda: (0, 0)),
        ],
        out_specs=(
            pl.BlockSpec((N, 1), lambda: (0, 0)),
            pl.BlockSpec((1, K), lambda: (0, 0)),
            pl.BlockSpec((K, DIM), lambda: (0, 0)),
            pl.BlockSpec((1, K), lambda: (0, 0)),
        ),
        scratch_shapes=[
            pltpu.VMEM((N, 1), jnp.int32),
            pltpu.VMEM((NTILE, K), jnp.float32),
        ],
    )

_scatter_call_dbg = pl.pallas_call(
        _scatter_kernel,
        grid=(NCH,),
        out_shape=jax.ShapeDtypeStruct((K, DIM), jnp.float32),
        in_specs=[
            pl.BlockSpec((1, 1, CHUNK), lambda g: (g, 0, 0),
                         memory_space=pltpu.MemorySpace.SMEM),
            pl.BlockSpec((1, K), lambda g: (0, 0),
                         memory_space=pltpu.MemorySpace.SMEM),
            pl.BlockSpec((CHUNK, DIM), lambda g: (g, 0)),
        ],
        out_specs=pl.BlockSpec((K, DIM), lambda g: (0, 0)),
        scratch_shapes=[
            pltpu.VMEM((K, DIM), jnp.float32),
            pltpu.VMEM((K, DIM), jnp.float32),
        ],
    )

_final_call_dbg = pl.pallas_call(
        _final_kernel,
        out_shape=jax.ShapeDtypeStruct((N, DIM), jnp.float32),
        in_specs=[
            pl.BlockSpec((N, DIM), lambda: (0, 0)),
            pl.BlockSpec((K, DIM), lambda: (0, 0)),
            pl.BlockSpec((K, 1), lambda: (0, 0)),
            pl.BlockSpec((K, DIM), lambda: (0, 0)),
        ],
        out_specs=pl.BlockSpec((N, DIM), lambda: (0, 0)),
    )

_W_ROW = jnp.array(WINDOW_ENDS + [-1] * (K - len(WINDOW_ENDS)),
                   jnp.float32).reshape(1, K)


@functools.partial(jax.jit, static_argnames=())
def kernel(embeddings, batch_id):
    del batch_id
    emb = embeddings
    w_row = _W_ROW
    assign_call = _assign_call_dbg
    scatter_call = _scatter_call_dbg
    final_call = _final_call_dbg

    def body(_, carry):
        sums, counts_col, cent = carry
        packed, counts_row, cent_new, ev = assign_call(
            emb, sums, counts_col, cent, w_row)
        sums_new = scatter_call(packed.reshape(NCH, 1, CHUNK), ev, emb)
        return sums_new, counts_row.reshape(K, 1), cent_new

    sums0 = emb[:K]
    counts0 = jnp.ones((K, 1), jnp.float32)
    cent0 = jnp.zeros((K, DIM), jnp.float32)
    sums, counts_col, cent = jax.lax.fori_loop(
        0, NITER, body, (sums0, counts0, cent0))

    return final_call(emb, sums, counts_col, cent)


# speedup vs baseline: 2.4407x; 1.2423x over previous
import functools

import jax
import jax.numpy as jnp
from jax.experimental import pallas as pl
from jax.experimental.pallas import tpu as pltpu

K = 64
NITER = 25
N = 16384
DIM = 128
CHUNK = 4096
TILE = 512
NCH = N // CHUNK
TPC = CHUNK // TILE
NTILE = N // TILE
DEFAULT = jax.lax.Precision.DEFAULT
HIGHEST = jax.lax.Precision.HIGHEST

_EH = [720, 1440, 2160] + list(range(2640, 7921, 480)) + [8192]
WINDOW_ENDS = _EH + [8192 + e for e in _EH[:-1]]


def _dot(a, b, dims, precision):
    return jax.lax.dot_general(
        a, b, dimension_numbers=(dims, ((), ())),
        precision=precision, preferred_element_type=jnp.float32)


def _rowsum_sq(x):
    sq = x * x
    acc8 = sq[:, 0:8]
    for g in range(1, 16):
        acc8 = acc8 + sq[:, 8 * g:8 * g + 8]
    a4 = acc8[:, 0:4] + acc8[:, 4:8]
    a2 = a4[:, 0:2] + a4[:, 2:4]
    return a2[:, 0:1] + a2[:, 1:2]


def _c2_row(cent, eye_k):
    c2_col = _rowsum_sq(cent)
    return _dot(c2_col, eye_k, ((0,), (0,)), HIGHEST)


def _argmin_col(d, iota_k):
    dmin = jnp.min(d, axis=1, keepdims=True)
    return jnp.min(jnp.where(d == dmin, iota_k, K), axis=1, keepdims=True)


def _argmax_col(d, iota_k):
    dmax = jnp.max(d, axis=1, keepdims=True)
    return jnp.min(jnp.where(d == dmax, iota_k, K), axis=1, keepdims=True)


def _centroids(sums, counts_col, cent_prev):
    return jnp.where(counts_col > 0.0,
                     sums / jnp.maximum(counts_col, 1.0), cent_prev)


def _assign_kernel(emb_ref, sums_ref, counts_ref, cent_prev_ref, wrow_ref,
                   packed_ref, counts_out_ref, cent_out_ref, ev_out_ref,
                   amin_ref, tilecnt_ref):
    cent = _centroids(sums_ref[...], counts_ref[...], cent_prev_ref[...])
    cent_out_ref[...] = cent
    iota_k = jax.lax.broadcasted_iota(jnp.int32, (1, K), 1)
    iota_kc = jax.lax.broadcasted_iota(jnp.int32, (TILE, K), 1)
    ey_r = jax.lax.broadcasted_iota(jnp.int32, (K, K), 0)
    ey_c = jax.lax.broadcasted_iota(jnp.int32, (K, K), 1)
    eye_k = (ey_r == ey_c).astype(jnp.float32)
    c2 = _c2_row(cent, eye_k)

    r_i = jax.lax.broadcasted_iota(jnp.int32, (TILE, TILE), 0)
    c_i = jax.lax.broadcasted_iota(jnp.int32, (TILE, TILE), 1)
    tri = (c_i < r_i).astype(jnp.float32)

    sel_t = jax.lax.broadcasted_iota(jnp.int32, (TPC, CHUNK), 0)
    sel_r = jax.lax.broadcasted_iota(jnp.int32, (TPC, CHUNK), 1)
    tile_sel = (sel_r // TILE == sel_t).astype(jnp.float32)

    def pass1(j, counts_row):
        x = emb_ref[pl.ds(j * CHUNK, CHUNK), :]
        x2c = _rowsum_sq(x)
        d = x2c + c2 - 2.0 * _dot(x, cent, ((1,), (1,)), DEFAULT)
        amin = _argmin_col(d, iota_k)
        amin_ref[pl.ds(j * CHUNK, CHUNK), :] = amin
        onehot = (amin == iota_k).astype(jnp.float32)
        counts_row = counts_row + _dot(
            jnp.ones((1, CHUNK), jnp.float32), onehot, ((1,), (0,)), DEFAULT)
        tilecnt_ref[pl.ds(j * TPC, TPC), :] = _dot(
            tile_sel, onehot, ((1,), (0,)), DEFAULT)
        return counts_row

    counts_row = jax.lax.fori_loop(0, NCH, pass1,
                                   jnp.zeros((1, K), jnp.float32))
    counts_out_ref[...] = counts_row

    cs_r = jax.lax.broadcasted_iota(jnp.int32, (K, K), 0)
    cs_c = jax.lax.broadcasted_iota(jnp.int32, (K, K), 1)
    triu = (cs_r < cs_c).astype(jnp.float32)
    offsets = _dot(counts_row, triu, ((1,), (0,)), HIGHEST)

    tr_r = jax.lax.broadcasted_iota(jnp.int32, (NTILE, NTILE), 0)
    tr_c = jax.lax.broadcasted_iota(jnp.int32, (NTILE, NTILE), 1)
    tri_t = (tr_c < tr_r).astype(jnp.float32)
    tile_prefix = _dot(tri_t, tilecnt_ref[...], ((1,), (0,)), HIGHEST)
    tilecnt_ref[...] = tile_prefix

    w_row = wrow_ref[...]
    rowvec = jax.lax.broadcasted_iota(jnp.int32, (1, TILE), 1).astype(jnp.float32)

    def pass2(t, ev_rows):
        amin = amin_ref[pl.ds(t * TILE, TILE), :]
        onehot = (amin == iota_kc).astype(jnp.float32)
        lrank = _dot(tri, onehot, ((1,), (0,)), DEFAULT)
        base = tilecnt_ref[pl.ds(t, 1), :]
        posf = lrank + base + offsets
        pos = jnp.sum(posf * onehot, axis=1, keepdims=True)
        packed_ref[pl.ds(t * TILE, TILE), :] = amin
        match = (pos == w_row).astype(jnp.float32)
        tf = jnp.float32(t * TILE)
        ev_rows = ev_rows + _dot(rowvec + tf, match, ((1,), (0,)), HIGHEST)
        return ev_rows

    ev_rows = jax.lax.fori_loop(0, NTILE, pass2,
                                jnp.zeros((1, K), jnp.float32))

    k_lane = jax.lax.broadcasted_iota(jnp.int32, (1, K), 1).astype(jnp.float32)
    is_pad = (w_row < 0.0).astype(jnp.float32)
    ev_rows = ev_rows * (1.0 - is_pad) + (100000.0 + k_lane) * is_pad
    evk = jnp.broadcast_to(ev_rows, (K, K))
    ev_diag_col = jnp.sum(evk * eye_k, axis=1, keepdims=True)
    cmp = (evk < ev_diag_col).astype(jnp.float32)
    rank_col = _dot(cmp, jnp.ones((K, 1), jnp.float32), ((1,), (0,)), DEFAULT)
    place = (rank_col == k_lane).astype(jnp.float32)
    ev_sorted = _dot(ev_rows, place, ((1,), (0,)), HIGHEST)
    ev_out_ref[...] = ev_sorted.astype(jnp.int32)


def _scatter_kernel(packed_ref, ev_ref, emb_ref, sums_ref,
                    partial_ref, total_ref):
    g = pl.program_id(0)

    @pl.when(g == 0)
    def _init():
        partial_ref[...] = jnp.zeros((K, DIM), jnp.float32)
        total_ref[...] = jnp.zeros((K, DIM), jnp.float32)

    def row_body(i, _):
        s = packed_ref[0, 0, i]
        partial_ref[pl.ds(s, 1), :] = (partial_ref[pl.ds(s, 1), :]
                                       + emb_ref[pl.ds(i, 1), :])
        return 0

    def row_body4(q, base):
        for u in range(4):
            row_body(base + 4 * q + u, 0)
        return base

    def ev_body(e, ptr):
        er = ev_ref[0, e] - g * CHUNK
        hi = jnp.clip(er, ptr, CHUNK)
        n4 = jax.lax.div(hi - ptr, 4)
        jax.lax.fori_loop(0, n4, row_body4, ptr)
        jax.lax.fori_loop(ptr + 4 * n4, hi, row_body, 0)

        @pl.when((er >= ptr) & (er < CHUNK))
        def _flush():
            s = packed_ref[0, 0, er]
            total_ref[pl.ds(s, 1), :] = (total_ref[pl.ds(s, 1), :]
                                         + partial_ref[pl.ds(s, 1), :])
            partial_ref[pl.ds(s, 1), :] = jnp.zeros((1, DIM), jnp.float32)

        return hi

    ptr = jax.lax.fori_loop(0, len(WINDOW_ENDS), ev_body, 0)
    n4 = jax.lax.div(CHUNK - ptr, 4)
    jax.lax.fori_loop(0, n4, row_body4, ptr)
    jax.lax.fori_loop(ptr + 4 * n4, CHUNK, row_body, 0)

    @pl.when(g == NCH - 1)
    def _fin():
        sums_ref[...] = total_ref[...] + partial_ref[...]


def _final_kernel(emb_ref, sums_ref, counts_ref, cent_prev_ref, out_ref):
    cent = _centroids(sums_ref[...], counts_ref[...], cent_prev_ref[...])
    iota_k = jax.lax.broadcasted_iota(jnp.int32, (1, K), 1)
    ey_r = jax.lax.broadcasted_iota(jnp.int32, (K, K), 0)
    ey_c = jax.lax.broadcasted_iota(jnp.int32, (K, K), 1)
    eye_k = (ey_r == ey_c).astype(jnp.float32)
    c2 = _c2_row(cent, eye_k)

    def chunk(j, _):
        x = emb_ref[pl.ds(j * CHUNK, CHUNK), :]
        x2c = _rowsum_sq(x)
        d = x2c + c2 - 2.0 * _dot(x, cent, ((1,), (1,)), DEFAULT)
        s = jnp.sqrt(jnp.maximum(d, 0.0))
        i1 = _argmax_col(s, iota_k)
        s_masked = jnp.where(iota_k == i1, -jnp.inf, s)
        i2 = _argmax_col(s_masked, iota_k)
        sel = (i2 == iota_k).astype(jnp.float32)
        out_ref[pl.ds(j * CHUNK, CHUNK), :] = _dot(
            sel, cent, ((1,), (0,)), HIGHEST)
        return 0

    jax.lax.fori_loop(0, NCH, chunk, 0)


_assign_call_dbg = pl.pallas_call(
        _assign_kernel,
        out_shape=(
            jax.ShapeDtypeStruct((N, 1), jnp.int32),
            jax.ShapeDtypeStruct((1, K), jnp.float32),
            jax.ShapeDtypeStruct((K, DIM), jnp.float32),
            jax.ShapeDtypeStruct((1, K), jnp.int32),
        ),
        in_specs=[
            pl.BlockSpec((N, DIM), lambda: (0, 0)),
            pl.BlockSpec((K, DIM), lambda: (0, 0)),
            pl.BlockSpec((K, 1), lambda: (0, 0)),
            pl.BlockSpec((K, DIM), lambda: (0, 0)),
            pl.BlockSpec((1, K), lambda: (0, 0)),
        ],
        out_specs=(
            pl.BlockSpec((N, 1), lambda: (0, 0)),
            pl.BlockSpec((1, K), lambda: (0, 0)),
            pl.BlockSpec((K, DIM), lambda: (0, 0)),
            pl.BlockSpec((1, K), lambda: (0, 0)),
        ),
        scratch_shapes=[
            pltpu.VMEM((N, 1), jnp.int32),
            pltpu.VMEM((NTILE, K), jnp.float32),
        ],
    )

_scatter_call_dbg = pl.pallas_call(
        _scatter_kernel,
        grid=(NCH,),
        out_shape=jax.ShapeDtypeStruct((K, DIM), jnp.float32),
        in_specs=[
            pl.BlockSpec((1, 1, CHUNK), lambda g: (g, 0, 0),
                         memory_space=pltpu.MemorySpace.SMEM),
            pl.BlockSpec((1, K), lambda g: (0, 0),
                         memory_space=pltpu.MemorySpace.SMEM),
            pl.BlockSpec((CHUNK, DIM), lambda g: (g, 0)),
        ],
        out_specs=pl.BlockSpec((K, DIM), lambda g: (0, 0)),
        scratch_shapes=[
            pltpu.VMEM((K, DIM), jnp.float32),
            pltpu.VMEM((K, DIM), jnp.float32),
        ],
    )

_final_call_dbg = pl.pallas_call(
        _final_kernel,
        out_shape=jax.ShapeDtypeStruct((N, DIM), jnp.float32),
        in_specs=[
            pl.BlockSpec((N, DIM), lambda: (0, 0)),
            pl.BlockSpec((K, DIM), lambda: (0, 0)),
            pl.BlockSpec((K, 1), lambda: (0, 0)),
            pl.BlockSpec((K, DIM), lambda: (0, 0)),
        ],
        out_specs=pl.BlockSpec((N, DIM), lambda: (0, 0)),
    )

_W_ROW = jnp.array(WINDOW_ENDS + [-1] * (K - len(WINDOW_ENDS)),
                   jnp.float32).reshape(1, K)


@functools.partial(jax.jit, static_argnames=())
def kernel(embeddings, batch_id):
    del batch_id
    emb = embeddings
    w_row = _W_ROW
    assign_call = _assign_call_dbg
    scatter_call = _scatter_call_dbg
    final_call = _final_call_dbg

    def body(_, carry):
        sums, counts_col, cent = carry
        packed, counts_row, cent_new, ev = assign_call(
            emb, sums, counts_col, cent, w_row)
        sums_new = scatter_call(packed.reshape(NCH, 1, CHUNK), ev, emb)
        return sums_new, counts_row.reshape(K, 1), cent_new

    sums0 = emb[:K]
    counts0 = jnp.ones((K, 1), jnp.float32)
    cent0 = jnp.zeros((K, DIM), jnp.float32)
    sums, counts_col, cent = jax.lax.fori_loop(
        0, NITER, body, (sums0, counts0, cent0))

    return final_call(emb, sums, counts_col, cent)


# speedup vs baseline: 2.5799x; 1.0570x over previous
import functools

import jax
import jax.numpy as jnp
from jax.experimental import pallas as pl
from jax.experimental.pallas import tpu as pltpu

K = 64
NITER = 25
N = 16384
DIM = 128
CHUNK = 4096
TILE = 512
NCH = N // CHUNK
TPC = CHUNK // TILE
NTILE = N // TILE
DEFAULT = jax.lax.Precision.DEFAULT
HIGHEST = jax.lax.Precision.HIGHEST

_EH = [720, 1440, 2160] + list(range(2640, 7921, 480)) + [8192]
WINDOW_ENDS = _EH + [8192 + e for e in _EH[:-1]]


def _dot(a, b, dims, precision):
    return jax.lax.dot_general(
        a, b, dimension_numbers=(dims, ((), ())),
        precision=precision, preferred_element_type=jnp.float32)


def _rowsum_sq(x):
    sq = x * x
    acc8 = sq[:, 0:8]
    for g in range(1, 16):
        acc8 = acc8 + sq[:, 8 * g:8 * g + 8]
    a4 = acc8[:, 0:4] + acc8[:, 4:8]
    a2 = a4[:, 0:2] + a4[:, 2:4]
    return a2[:, 0:1] + a2[:, 1:2]


def _c2_row(cent, eye_k):
    c2_col = _rowsum_sq(cent)
    return _dot(c2_col, eye_k, ((0,), (0,)), HIGHEST)


def _argmin_col(d, iota_k):
    dmin = jnp.min(d, axis=1, keepdims=True)
    return jnp.min(jnp.where(d == dmin, iota_k, K), axis=1, keepdims=True)


def _argmax_col(d, iota_k):
    dmax = jnp.max(d, axis=1, keepdims=True)
    return jnp.min(jnp.where(d == dmax, iota_k, K), axis=1, keepdims=True)


def _centroids(sums, counts_col, cent_prev):
    return jnp.where(counts_col > 0.0,
                     sums / jnp.maximum(counts_col, 1.0), cent_prev)


def _assign_kernel(emb_ref, sums_ref, counts_ref, cent_prev_ref, wrow_ref,
                   packed_ref, counts_out_ref, cent_out_ref, ev_out_ref,
                   amin_ref, tilecnt_ref):
    cent = _centroids(sums_ref[...], counts_ref[...], cent_prev_ref[...])
    cent_out_ref[...] = cent
    iota_k = jax.lax.broadcasted_iota(jnp.int32, (1, K), 1)
    iota_kc = jax.lax.broadcasted_iota(jnp.int32, (TILE, K), 1)
    ey_r = jax.lax.broadcasted_iota(jnp.int32, (K, K), 0)
    ey_c = jax.lax.broadcasted_iota(jnp.int32, (K, K), 1)
    eye_k = (ey_r == ey_c).astype(jnp.float32)
    c2 = _c2_row(cent, eye_k)

    r_i = jax.lax.broadcasted_iota(jnp.int32, (TILE, TILE), 0)
    c_i = jax.lax.broadcasted_iota(jnp.int32, (TILE, TILE), 1)
    tri = (c_i < r_i).astype(jnp.float32)

    sel_t = jax.lax.broadcasted_iota(jnp.int32, (TPC, CHUNK), 0)
    sel_r = jax.lax.broadcasted_iota(jnp.int32, (TPC, CHUNK), 1)
    tile_sel = (sel_r // TILE == sel_t).astype(jnp.float32)

    def pass1(j, counts_row):
        x = emb_ref[pl.ds(j * CHUNK, CHUNK), :]
        x2c = _rowsum_sq(x)
        d = x2c + c2 - 2.0 * _dot(x, cent, ((1,), (1,)), DEFAULT)
        amin = _argmin_col(d, iota_k)
        amin_ref[pl.ds(j * CHUNK, CHUNK), :] = amin
        onehot = (amin == iota_k).astype(jnp.float32)
        counts_row = counts_row + _dot(
            jnp.ones((1, CHUNK), jnp.float32), onehot, ((1,), (0,)), DEFAULT)
        tilecnt_ref[pl.ds(j * TPC, TPC), :] = _dot(
            tile_sel, onehot, ((1,), (0,)), DEFAULT)
        return counts_row

    counts_row = jax.lax.fori_loop(0, NCH, pass1,
                                   jnp.zeros((1, K), jnp.float32))
    counts_out_ref[...] = counts_row

    cs_r = jax.lax.broadcasted_iota(jnp.int32, (K, K), 0)
    cs_c = jax.lax.broadcasted_iota(jnp.int32, (K, K), 1)
    triu = (cs_r < cs_c).astype(jnp.float32)
    offsets = _dot(counts_row, triu, ((1,), (0,)), HIGHEST)

    tr_r = jax.lax.broadcasted_iota(jnp.int32, (NTILE, NTILE), 0)
    tr_c = jax.lax.broadcasted_iota(jnp.int32, (NTILE, NTILE), 1)
    tri_t = (tr_c < tr_r).astype(jnp.float32)
    tile_prefix = _dot(tri_t, tilecnt_ref[...], ((1,), (0,)), HIGHEST)
    tilecnt_ref[...] = tile_prefix

    w_row = wrow_ref[...]
    rowvec = jax.lax.broadcasted_iota(jnp.int32, (1, TILE), 1).astype(jnp.float32)

    def pass2(t, ev_rows):
        amin = amin_ref[pl.ds(t * TILE, TILE), :]
        onehot = (amin == iota_kc).astype(jnp.float32)
        lrank = _dot(tri, onehot, ((1,), (0,)), DEFAULT)
        base = tilecnt_ref[pl.ds(t, 1), :]
        posf = lrank + base + offsets
        pos = jnp.sum(posf * onehot, axis=1, keepdims=True)
        packed_ref[pl.ds(t * TILE, TILE), :] = amin
        match = (pos == w_row).astype(jnp.float32)
        tf = jnp.float32(t * TILE)
        ev_rows = ev_rows + _dot(rowvec + tf, match, ((1,), (0,)), HIGHEST)
        return ev_rows

    ev_rows = jax.lax.fori_loop(0, NTILE, pass2,
                                jnp.zeros((1, K), jnp.float32))

    k_lane = jax.lax.broadcasted_iota(jnp.int32, (1, K), 1).astype(jnp.float32)
    is_pad = (w_row < 0.0).astype(jnp.float32)
    ev_rows = ev_rows * (1.0 - is_pad) + (100000.0 + k_lane) * is_pad
    evk = jnp.broadcast_to(ev_rows, (K, K))
    ev_diag_col = jnp.sum(evk * eye_k, axis=1, keepdims=True)
    cmp = (evk < ev_diag_col).astype(jnp.float32)
    rank_col = _dot(cmp, jnp.ones((K, 1), jnp.float32), ((1,), (0,)), DEFAULT)
    place = (rank_col == k_lane).astype(jnp.float32)
    ev_sorted = _dot(ev_rows, place, ((1,), (0,)), HIGHEST)
    ev_out_ref[...] = ev_sorted.astype(jnp.int32)


def _scatter_kernel(packed_ref, ev_ref, emb_ref, sums_ref,
                    partial_ref, total_ref):
    g = pl.program_id(0)

    @pl.when(g == 0)
    def _init():
        partial_ref[...] = jnp.zeros((K, DIM), jnp.float32)
        total_ref[...] = jnp.zeros((K, DIM), jnp.float32)

    def row_body(i, _):
        s = packed_ref[0, 0, i]
        partial_ref[pl.ds(s, 1), :] = (partial_ref[pl.ds(s, 1), :]
                                       + emb_ref[pl.ds(i, 1), :])
        return 0

    def row_body4(q, base):
        for u in range(8):
            row_body(base + 8 * q + u, 0)
        return base

    def ev_body(e, ptr):
        er = ev_ref[0, e] - g * CHUNK
        hi = jnp.clip(er, ptr, CHUNK)
        n4 = jax.lax.div(hi - ptr, 8)
        jax.lax.fori_loop(0, n4, row_body4, ptr)
        jax.lax.fori_loop(ptr + 8 * n4, hi, row_body, 0)

        @pl.when((er >= ptr) & (er < CHUNK))
        def _flush():
            s = packed_ref[0, 0, er]
            total_ref[pl.ds(s, 1), :] = (total_ref[pl.ds(s, 1), :]
                                         + partial_ref[pl.ds(s, 1), :])
            partial_ref[pl.ds(s, 1), :] = jnp.zeros((1, DIM), jnp.float32)

        return hi

    ptr = jax.lax.fori_loop(0, len(WINDOW_ENDS), ev_body, 0)
    n4 = jax.lax.div(CHUNK - ptr, 8)
    jax.lax.fori_loop(0, n4, row_body4, ptr)
    jax.lax.fori_loop(ptr + 8 * n4, CHUNK, row_body, 0)

    @pl.when(g == NCH - 1)
    def _fin():
        sums_ref[...] = total_ref[...] + partial_ref[...]


def _final_kernel(emb_ref, sums_ref, counts_ref, cent_prev_ref, out_ref):
    cent = _centroids(sums_ref[...], counts_ref[...], cent_prev_ref[...])
    iota_k = jax.lax.broadcasted_iota(jnp.int32, (1, K), 1)
    ey_r = jax.lax.broadcasted_iota(jnp.int32, (K, K), 0)
    ey_c = jax.lax.broadcasted_iota(jnp.int32, (K, K), 1)
    eye_k = (ey_r == ey_c).astype(jnp.float32)
    c2 = _c2_row(cent, eye_k)

    def chunk(j, _):
        x = emb_ref[pl.ds(j * CHUNK, CHUNK), :]
        x2c = _rowsum_sq(x)
        d = x2c + c2 - 2.0 * _dot(x, cent, ((1,), (1,)), DEFAULT)
        s = jnp.sqrt(jnp.maximum(d, 0.0))
        i1 = _argmax_col(s, iota_k)
        s_masked = jnp.where(iota_k == i1, -jnp.inf, s)
        i2 = _argmax_col(s_masked, iota_k)
        sel = (i2 == iota_k).astype(jnp.float32)
        out_ref[pl.ds(j * CHUNK, CHUNK), :] = _dot(
            sel, cent, ((1,), (0,)), HIGHEST)
        return 0

    jax.lax.fori_loop(0, NCH, chunk, 0)


_assign_call_dbg = pl.pallas_call(
        _assign_kernel,
        out_shape=(
            jax.ShapeDtypeStruct((N, 1), jnp.int32),
            jax.ShapeDtypeStruct((1, K), jnp.float32),
            jax.ShapeDtypeStruct((K, DIM), jnp.float32),
            jax.ShapeDtypeStruct((1, K), jnp.int32),
        ),
        in_specs=[
            pl.BlockSpec((N, DIM), lambda: (0, 0)),
            pl.BlockSpec((K, DIM), lambda: (0, 0)),
            pl.BlockSpec((K, 1), lambda: (0, 0)),
            pl.BlockSpec((K, DIM), lambda: (0, 0)),
            pl.BlockSpec((1, K), lambda: (0, 0)),
        ],
        out_specs=(
            pl.BlockSpec((N, 1), lambda: (0, 0)),
            pl.BlockSpec((1, K), lambda: (0, 0)),
            pl.BlockSpec((K, DIM), lambda: (0, 0)),
            pl.BlockSpec((1, K), lambda: (0, 0)),
        ),
        scratch_shapes=[
            pltpu.VMEM((N, 1), jnp.int32),
            pltpu.VMEM((NTILE, K), jnp.float32),
        ],
    )

_scatter_call_dbg = pl.pallas_call(
        _scatter_kernel,
        grid=(NCH,),
        out_shape=jax.ShapeDtypeStruct((K, DIM), jnp.float32),
        in_specs=[
            pl.BlockSpec((1, 1, CHUNK), lambda g: (g, 0, 0),
                         memory_space=pltpu.MemorySpace.SMEM),
            pl.BlockSpec((1, K), lambda g: (0, 0),
                         memory_space=pltpu.MemorySpace.SMEM),
            pl.BlockSpec((CHUNK, DIM), lambda g: (g, 0)),
        ],
        out_specs=pl.BlockSpec((K, DIM), lambda g: (0, 0)),
        scratch_shapes=[
            pltpu.VMEM((K, DIM), jnp.float32),
            pltpu.VMEM((K, DIM), jnp.float32),
        ],
    )

_final_call_dbg = pl.pallas_call(
        _final_kernel,
        out_shape=jax.ShapeDtypeStruct((N, DIM), jnp.float32),
        in_specs=[
            pl.BlockSpec((N, DIM), lambda: (0, 0)),
            pl.BlockSpec((K, DIM), lambda: (0, 0)),
            pl.BlockSpec((K, 1), lambda: (0, 0)),
            pl.BlockSpec((K, DIM), lambda: (0, 0)),
        ],
        out_specs=pl.BlockSpec((N, DIM), lambda: (0, 0)),
    )

_W_ROW = jnp.array(WINDOW_ENDS + [-1] * (K - len(WINDOW_ENDS)),
                   jnp.float32).reshape(1, K)


@functools.partial(jax.jit, static_argnames=())
def kernel(embeddings, batch_id):
    del batch_id
    emb = embeddings
    w_row = _W_ROW
    assign_call = _assign_call_dbg
    scatter_call = _scatter_call_dbg
    final_call = _final_call_dbg

    def body(_, carry):
        sums, counts_col, cent = carry
        packed, counts_row, cent_new, ev = assign_call(
            emb, sums, counts_col, cent, w_row)
        sums_new = scatter_call(packed.reshape(NCH, 1, CHUNK), ev, emb)
        return sums_new, counts_row.reshape(K, 1), cent_new

    sums0 = emb[:K]
    counts0 = jnp.ones((K, 1), jnp.float32)
    cent0 = jnp.zeros((K, DIM), jnp.float32)
    sums, counts_col, cent = jax.lax.fori_loop(
        0, NITER, body, (sums0, counts0, cent0))

    return final_call(emb, sums, counts_col, cent)
